# Initial kernel scaffold; baseline (speedup 1.0000x reference)
import jax
import jax.numpy as jnp
from jax.experimental import pallas as pl


def kernel(x, edge_attr, params, edge_index, batch):
    raise NotImplementedError("write your pallas kernel here")


# speedup vs baseline: 1.5435x; 1.5435x over previous
import functools

import jax
import jax.numpy as jnp
from jax import lax
from jax.experimental import pallas as pl
from jax.experimental.pallas import tpu as pltpu
from jax.experimental.pallas import tpu_sc as plsc

N = 10000
E = 320000
F_IN = 128
H = 256
K = 5
DIM = 4
OUT = 2
BG = 64
HH = H // 2
KH = K * HH
NC = 2
NS = 16
NW = NC * NS
EPT = E // NW
CH = 40
NCHUNK = EPT // CH
SL = 632
SLL = N - (NS - 1) * SL
NB = 1000
EB = 20000
F32 = jnp.float32


def _sc_conv_body(with_deg, *refs):
    if with_deg:
        (hga, hgb, src, dst, gauss, zrow, aggp, degp,
         srcv, dstv, gaussv, rows, msg, agg_sh, sem) = refs
    else:
        (hga, hgb, src, dst, gauss, zrow, aggp,
         srcv, dstv, gaussv, rows, msg, agg_sh, sem) = refs
    c = lax.axis_index("c")
    s = lax.axis_index("s")
    ebase = (c * NS + s) * EPT

    def _per_slice(fn):
        @pl.when(s < NS - 1)
        def _():
            fn(pl.ds(s * SL, SL))

        @pl.when(s == NS - 1)
        def _():
            fn(pl.ds((NS - 1) * SL, SLL))

    _per_slice(lambda sl: pltpu.sync_copy(zrow.at[sl], agg_sh.at[sl]))
    plsc.subcore_barrier()

    for half in range(2):
        table = hga if half == 0 else hgb

        def _chunk(ci, carry):
            eb = ebase + ci * CH
            pltpu.sync_copy(src.at[pl.ds(eb, CH)], srcv)
            pltpu.sync_copy(dst.at[pl.ds(eb, CH)], dstv)
            pltpu.sync_copy(gauss.at[pl.ds(eb, CH)], gaussv)
            pltpu.async_copy(table.at[srcv], rows, sem).wait()

            def _edge(e, ecarry):
                gv = gaussv[e, :]
                for j in range(HH // 16):
                    acc = jnp.zeros((16,), F32)
                    for k in range(K):
                        gk = jnp.full((16,), gv[k], F32)
                        acc = acc + gk * rows[e, pl.ds(k * HH + j * 16, 16)]
                    msg[e, pl.ds(j * 16, 16)] = acc
                return ecarry
            lax.fori_loop(0, CH, _edge, 0)

            pltpu.sync_copy(msg, agg_sh.at[dstv], add=True)
            return carry
        lax.fori_loop(0, NCHUNK, _chunk, 0)
        plsc.subcore_barrier()

        _per_slice(lambda sl: pltpu.sync_copy(agg_sh.at[sl], aggp.at[c, half, sl]))
        if half == 0 or with_deg:
            _per_slice(lambda sl: pltpu.sync_copy(zrow.at[sl], agg_sh.at[sl]))
            plsc.subcore_barrier()

    if with_deg:
        def _fill_ones(i, carry):
            for j in range(HH // 16):
                msg[i, pl.ds(j * 16, 16)] = jnp.ones((16,), F32)
            return carry
        lax.fori_loop(0, CH, _fill_ones, 0)

        def _dchunk(ci, carry):
            eb = ebase + ci * CH
            pltpu.sync_copy(dst.at[pl.ds(eb, CH)], dstv)
            pltpu.sync_copy(msg, agg_sh.at[dstv], add=True)
            return carry
        lax.fori_loop(0, NCHUNK, _dchunk, 0)
        plsc.subcore_barrier()
        _per_slice(lambda sl: pltpu.sync_copy(agg_sh.at[sl], degp.at[c, sl]))


def _make_sc_conv(with_deg):
    mesh = plsc.VectorSubcoreMesh(core_axis_name="c", subcore_axis_name="s")
    out_type = [jax.ShapeDtypeStruct((NC, 2, N, HH), F32)]
    if with_deg:
        out_type.append(jax.ShapeDtypeStruct((NC, N, HH), F32))
    return pl.kernel(
        functools.partial(_sc_conv_body, with_deg),
        out_type=out_type if with_deg else out_type[0],
        mesh=mesh,
        scratch_types=[
            pltpu.VMEM((CH,), jnp.int32),
            pltpu.VMEM((CH,), jnp.int32),
            pltpu.VMEM((CH, 16), F32),
            pltpu.VMEM((CH, KH), F32),
            pltpu.VMEM((CH, HH), F32),
            pltpu.VMEM_SHARED((N, HH), F32),
            pltpu.SemaphoreType.DMA,
        ],
    )


_sc_conv_deg = _make_sc_conv(True)
_sc_conv = _make_sc_conv(False)


def _gauss_body(ea_ref, m_ref, c_ref, out_ref):
    ea = ea_ref[...]
    feat = jnp.concatenate([ea * ea, ea], axis=1)
    r = jnp.exp(jnp.dot(feat, m_ref[0], preferred_element_type=F32) + c_ref[0])
    out_ref[0] = jnp.concatenate([r, jnp.zeros((EB, 16 - K), F32)], axis=1)


def _gauss_call(edge_attr, m, c):
    return pl.pallas_call(
        _gauss_body,
        grid=(3, E // EB),
        in_specs=[
            pl.BlockSpec((EB, DIM), lambda i, j: (j, 0)),
            pl.BlockSpec((1, 2 * DIM, K), lambda i, j: (i, 0, 0)),
            pl.BlockSpec((1, 1, K), lambda i, j: (i, 0, 0)),
        ],
        out_specs=pl.BlockSpec((1, EB, 16), lambda i, j: (i, j, 0)),
        out_shape=jax.ShapeDtypeStruct((3, E, 16), F32),
    )(edge_attr, m, c)


def _prep0_body(x_ref, w_ref, b_ref, ga_ref, gb_ref, rt_ref,
                hga_ref, hgb_ref, hrt_ref):
    h = jnp.dot(x_ref[...], w_ref[...], preferred_element_type=F32) + b_ref[...]
    hga_ref[...] = jnp.dot(h, ga_ref[...], preferred_element_type=F32)
    hgb_ref[...] = jnp.dot(h, gb_ref[...], preferred_element_type=F32)
    hrt_ref[...] = jnp.dot(h, rt_ref[...], preferred_element_type=F32)


def _prep0_call(x, w, b, ga, gb, rt):
    return pl.pallas_call(
        _prep0_body,
        grid=(N // NB,),
        in_specs=[
            pl.BlockSpec((NB, F_IN), lambda i: (i, 0)),
            pl.BlockSpec((F_IN, H), lambda i: (0, 0)),
            pl.BlockSpec((1, H), lambda i: (0, 0)),
            pl.BlockSpec((H, KH), lambda i: (0, 0)),
            pl.BlockSpec((H, KH), lambda i: (0, 0)),
            pl.BlockSpec((H, H), lambda i: (0, 0)),
        ],
        out_specs=[
            pl.BlockSpec((NB, KH), lambda i: (i, 0)),
            pl.BlockSpec((NB, KH), lambda i: (i, 0)),
            pl.BlockSpec((NB, H), lambda i: (i, 0)),
        ],
        out_shape=[
            jax.ShapeDtypeStruct((N, KH), F32),
            jax.ShapeDtypeStruct((N, KH), F32),
            jax.ShapeDtypeStruct((N, H), F32),
        ],
    )(x, w, b, ga, gb, rt)


def _combine(aggp_ref, degp_ref, hrt_ref, bias_ref):
    deg = degp_ref[0, :, :1] + degp_ref[1, :, :1]
    deg = jnp.maximum(deg, 1.0)
    a0 = aggp_ref[0, 0] + aggp_ref[1, 0]
    a1 = aggp_ref[0, 1] + aggp_ref[1, 1]
    agg = jnp.concatenate([a0, a1], axis=1)
    return agg / deg + hrt_ref[...] + bias_ref[...]


def _prep_mid_body(aggp_ref, degp_ref, hrt_ref, bias_ref,
                   ga_ref, gb_ref, rt_ref, hga_ref, hgb_ref, hrt_o_ref):
    ht = jnp.tanh(_combine(aggp_ref, degp_ref, hrt_ref, bias_ref))
    hga_ref[...] = jnp.dot(ht, ga_ref[...], preferred_element_type=F32)
    hgb_ref[...] = jnp.dot(ht, gb_ref[...], preferred_element_type=F32)
    hrt_o_ref[...] = jnp.dot(ht, rt_ref[...], preferred_element_type=F32)


def _prep_mid_call(aggp, degp, hrt, bias, ga, gb, rt):
    return pl.pallas_call(
        _prep_mid_body,
        grid=(N // NB,),
        in_specs=[
            pl.BlockSpec((NC, 2, NB, HH), lambda i: (0, 0, i, 0)),
            pl.BlockSpec((NC, NB, HH), lambda i: (0, i, 0)),
            pl.BlockSpec((NB, H), lambda i: (i, 0)),
            pl.BlockSpec((1, H), lambda i: (0, 0)),
            pl.BlockSpec((H, KH), lambda i: (0, 0)),
            pl.BlockSpec((H, KH), lambda i: (0, 0)),
            pl.BlockSpec((H, H), lambda i: (0, 0)),
        ],
        out_specs=[
            pl.BlockSpec((NB, KH), lambda i: (i, 0)),
            pl.BlockSpec((NB, KH), lambda i: (i, 0)),
            pl.BlockSpec((NB, H), lambda i: (i, 0)),
        ],
        out_shape=[
            jax.ShapeDtypeStruct((N, KH), F32),
            jax.ShapeDtypeStruct((N, KH), F32),
            jax.ShapeDtypeStruct((N, H), F32),
        ],
    )(aggp, degp, hrt, bias, ga, gb, rt)


def _pool_body(aggp_ref, degp_ref, hrt_ref, bias_ref, batch_ref, out_ref):
    i = pl.program_id(0)
    h3 = _combine(aggp_ref, degp_ref, hrt_ref, bias_ref)
    b = batch_ref[0, 0]
    onehot = (lax.broadcasted_iota(jnp.int32, (BG, NB), 0)
              == b[None, :]).astype(F32)
    part = jnp.dot(onehot, h3, preferred_element_type=F32)

    @pl.when(i == 0)
    def _():
        out_ref[...] = jnp.zeros_like(out_ref)
    out_ref[...] += part


def _pool_call(aggp, degp, hrt, bias, batch3):
    return pl.pallas_call(
        _pool_body,
        grid=(N // NB,),
        in_specs=[
            pl.BlockSpec((NC, 2, NB, HH), lambda i: (0, 0, i, 0)),
            pl.BlockSpec((NC, NB, HH), lambda i: (0, i, 0)),
            pl.BlockSpec((NB, H), lambda i: (i, 0)),
            pl.BlockSpec((1, H), lambda i: (0, 0)),
            pl.BlockSpec((1, 1, NB), lambda i: (i, 0, 0)),
        ],
        out_specs=pl.BlockSpec((BG, H), lambda i: (0, 0)),
        out_shape=jax.ShapeDtypeStruct((BG, H), F32),
    )(aggp, degp, hrt, bias, batch3)


def _bn(h, gamma, beta):
    m = jnp.mean(h, axis=0, keepdims=True)
    v = jnp.mean((h - m) ** 2, axis=0, keepdims=True)
    return gamma * (h - m) * lax.rsqrt(v + 1e-5) + beta


def _head_body(pooled_ref, lw_ref, lb_ref, gam_ref, bet_ref, wo_ref, bo_ref,
               out_ref):
    h = pooled_ref[...]
    for i in range(3):
        h = _bn(h, gam_ref[i], bet_ref[i])
        h = jnp.dot(h, lw_ref[i], preferred_element_type=F32) + lb_ref[i]
        h = jnp.tanh(h)
    h = _bn(h, gam_ref[3], bet_ref[3])
    out_ref[...] = jnp.dot(h, wo_ref[...], preferred_element_type=F32) + bo_ref[...]


def _head_call(pooled, lw, lb, gam, bet, wo, bo):
    full = lambda *s: pl.BlockSpec(s, lambda: tuple(0 for _ in s))
    return pl.pallas_call(
        _head_body,
        in_specs=[
            full(BG, H),
            full(3, H, H),
            full(3, 1, H),
            full(4, 1, H),
            full(4, 1, H),
            full(H, OUT),
            full(1, OUT),
        ],
        out_specs=full(BG, OUT),
        out_shape=jax.ShapeDtypeStruct((BG, OUT), F32),
    )(pooled, lw, lb, gam, bet, wo, bo)


def kernel(x, edge_attr, params, edge_index, batch):
    p = params
    src = edge_index[0]
    dst = edge_index[1]

    mu = p["conv_mu"]
    sg = p["conv_sigma"]
    a = 1.0 / (1e-15 + sg * sg)
    m = jnp.concatenate([(-0.5 * a).transpose(0, 2, 1),
                         (mu * a).transpose(0, 2, 1)], axis=1)
    cc = (-0.5 * jnp.sum(mu * mu * a, axis=-1))[:, None, :]
    gauss = _gauss_call(edge_attr, m, cc)

    g4 = p["conv_g"].reshape(3, H, K, H)
    ga = g4[:, :, :, :HH].reshape(3, H, KH)
    gb = g4[:, :, :, HH:].reshape(3, H, KH)

    zrow = jnp.zeros((N, HH), F32)

    hga, hgb, hrt = _prep0_call(x, p["W_ne"], p["b_ne"].reshape(1, H),
                                ga[0], gb[0], p["conv_root"][0])
    aggp, degp = _sc_conv_deg(hga, hgb, src, dst, gauss[0], zrow)

    for i in (1, 2):
        hga, hgb, hrt = _prep_mid_call(
            aggp, degp, hrt, p["conv_bias"][i - 1].reshape(1, H),
            ga[i], gb[i], p["conv_root"][i])
        aggp = _sc_conv(hga, hgb, src, dst, gauss[i], zrow)

    pooled = _pool_call(aggp, degp, hrt, p["conv_bias"][2].reshape(1, H),
                        batch.reshape(N // NB, 1, NB))

    return _head_call(pooled, p["lin_w"], p["lin_b"].reshape(3, 1, H),
                      p["bn_gamma"].reshape(4, 1, H),
                      p["bn_beta"].reshape(4, 1, H),
                      p["W_out"], p["b_out"].reshape(1, OUT))


# speedup vs baseline: 2.7719x; 1.7959x over previous
import functools

import jax
import jax.numpy as jnp
from jax import lax
from jax.experimental import pallas as pl
from jax.experimental.pallas import tpu as pltpu
from jax.experimental.pallas import tpu_sc as plsc

N = 10000
E = 320000
F_IN = 128
H = 256
K = 5
DIM = 4
OUT = 2
BG = 64
HH = H // 2
KH = K * HH
NC = 2
NS = 16
NW = NC * NS
CH = 16
UNROLL = 4
NCHUNK = 628
EPT = NCHUNK * CH
EP = EPT * NW
SL = 632
SLL = N - (NS - 1) * SL
NB = 1000
EB = EP // 16
F32 = jnp.float32


def _sc_conv_body(with_deg, *refs):
    if with_deg:
        (src3, dst3, gauss3, hga, hgb, zrow, aggp, degp,
         srcv, dstv, gaussv, rows, msg,
         agg_sh, f0, f1, f2, f3, r0, r1, s0, s1) = refs
    else:
        (src3, dst3, gauss3, hga, hgb, zrow, aggp,
         srcv, dstv, gaussv, rows, msg,
         agg_sh, f0, f1, f2, f3, r0, r1, s0, s1) = refs
    fsem = (f0, f1, f2, f3)
    rsem = (r0, r1)
    ssem = (s0, s1)
    c = lax.axis_index("c")
    s = lax.axis_index("s")

    def _per_slice(fn):
        @pl.when(s < NS - 1)
        def _():
            fn(pl.ds(s * SL, SL))

        @pl.when(s == NS - 1)
        def _():
            fn(pl.ds((NS - 1) * SL, SLL))

    cbase = (c * NS + s) * NCHUNK

    def _fetch_idx(ci, j, with_data):
        pltpu.async_copy(dst3.at[cbase + ci], dstv.at[j], fsem[j])
        if with_data:
            pltpu.async_copy(src3.at[cbase + ci], srcv.at[j], fsem[j])
            pltpu.async_copy(gauss3.at[cbase + ci], gaussv.at[j], fsem[j])

    def _wait_idx(j, with_data):
        pltpu.make_async_copy(dst3.at[cbase], dstv.at[j], fsem[j]).wait()
        if with_data:
            pltpu.make_async_copy(src3.at[cbase], srcv.at[j], fsem[j]).wait()
            pltpu.make_async_copy(gauss3.at[cbase], gaussv.at[j],
                                  fsem[j]).wait()

    def _issue_gather(j, b):
        pltpu.async_copy(_table[0].at[srcv.at[j, 0]], rows.at[b], rsem[b])

    def _wait_gather(j, b):
        pltpu.make_async_copy(_table[0].at[srcv.at[j, 0]], rows.at[b],
                              rsem[b]).wait()

    def _issue_scatter(j, b):
        pltpu.async_copy(msg.at[b], agg_sh.at[dstv.at[j, 0]], ssem[b],
                         add=True)

    def _wait_scatter(j, b):
        pltpu.make_async_copy(msg.at[b], agg_sh.at[dstv.at[j, 0]],
                              ssem[b]).wait()

    _per_slice(lambda sl: pltpu.sync_copy(zrow.at[sl], agg_sh.at[sl]))
    plsc.subcore_barrier()

    for half in range(2):
        _table = (hga if half == 0 else hgb,)
        _fetch_idx(0, 0, True)
        _fetch_idx(1, 1, True)
        _wait_idx(0, True)
        _issue_gather(0, 0)

        def _quad(cq, carry):
            for u in range(UNROLL):
                ci = cq * UNROLL + u
                b2 = u % 2
                b4 = u

                @pl.when(ci + 1 < NCHUNK)
                def _():
                    _wait_idx((u + 1) % UNROLL, True)
                    _issue_gather((u + 1) % UNROLL, 1 - b2)

                @pl.when(ci >= 2)
                def _():
                    _wait_scatter(b4, b2)

                @pl.when(ci + 2 < NCHUNK)
                def _():
                    _fetch_idx(ci + 2, (u + 2) % UNROLL, True)

                _wait_gather(b4, b2)

                def _edge(e, ecarry):
                    gv = gaussv[b4, e, :]
                    for j in range(HH // 16):
                        acc = jnp.zeros((16,), F32)
                        for k in range(K):
                            gk = jnp.full((16,), gv[k], F32)
                            acc = acc + gk * rows[b2, e,
                                                  pl.ds(k * HH + j * 16, 16)]
                        msg[b2, e, pl.ds(j * 16, 16)] = acc
                    return ecarry
                lax.fori_loop(0, CH, _edge, 0)

                _issue_scatter(b4, b2)
            return carry
        lax.fori_loop(0, NCHUNK // UNROLL, _quad, 0)
        _wait_scatter(0, 0)
        _wait_scatter(1, 1)
        plsc.subcore_barrier()

        _per_slice(lambda sl: pltpu.sync_copy(agg_sh.at[sl], aggp.at[c, half, sl]))
        if half == 0 or with_deg:
            _per_slice(lambda sl: pltpu.sync_copy(zrow.at[sl], agg_sh.at[sl]))
            plsc.subcore_barrier()

    if with_deg:
        def _fill_ones(i, carry):
            for b in range(2):
                for j in range(HH // 16):
                    msg[b, i, pl.ds(j * 16, 16)] = jnp.ones((16,), F32)
            return carry
        lax.fori_loop(0, CH, _fill_ones, 0)

        _fetch_idx(0, 0, False)
        _fetch_idx(1, 1, False)

        def _dquad(cq, carry):
            for u in range(UNROLL):
                ci = cq * UNROLL + u
                b2 = u % 2
                b4 = u

                @pl.when(ci >= 2)
                def _():
                    _wait_scatter(b4, b2)

                @pl.when(ci + 2 < NCHUNK)
                def _():
                    _fetch_idx(ci + 2, (u + 2) % UNROLL, False)

                _wait_idx(b4, False)
                _issue_scatter(b4, b2)
            return carry
        lax.fori_loop(0, NCHUNK // UNROLL, _dquad, 0)
        _wait_scatter(0, 0)
        _wait_scatter(1, 1)
        plsc.subcore_barrier()
        _per_slice(lambda sl: pltpu.sync_copy(agg_sh.at[sl], degp.at[c, sl]))


def _make_sc_conv(with_deg):
    mesh = plsc.VectorSubcoreMesh(core_axis_name="c", subcore_axis_name="s")
    out_type = [jax.ShapeDtypeStruct((NC, 2, N, HH), F32)]
    if with_deg:
        out_type.append(jax.ShapeDtypeStruct((NC, N, HH), F32))
    return pl.kernel(
        functools.partial(_sc_conv_body, with_deg),
        out_type=out_type if with_deg else out_type[0],
        mesh=mesh,
        scratch_types=[
            pltpu.VMEM((UNROLL, 1, CH), jnp.int32),
            pltpu.VMEM((UNROLL, 1, CH), jnp.int32),
            pltpu.VMEM((UNROLL, CH, 16), F32),
            pltpu.VMEM((2, CH, KH), F32),
            pltpu.VMEM((2, CH, HH), F32),
            pltpu.VMEM_SHARED((N + 8, HH), F32),
        ] + [pltpu.SemaphoreType.DMA] * 8,
    )


_sc_conv_deg = _make_sc_conv(True)
_sc_conv = _make_sc_conv(False)


def _gauss_body(ea_ref, m_ref, c_ref, out_ref):
    j = pl.program_id(1)
    ea = ea_ref[...]
    feat = jnp.concatenate([ea * ea, ea], axis=1)
    r = jnp.exp(jnp.dot(feat, m_ref[0], preferred_element_type=F32) + c_ref[0])
    valid = (lax.broadcasted_iota(jnp.int32, (EB, 16), 0) + j * EB < E)
    valid &= lax.broadcasted_iota(jnp.int32, (EB, 16), 1) < K
    r16 = jnp.concatenate([r, jnp.zeros((EB, 16 - K), F32)], axis=1)
    out_ref[0] = jnp.where(valid, r16, 0.0)


def _gauss_call(edge_attr_p, m, c):
    return pl.pallas_call(
        _gauss_body,
        grid=(3, EP // EB),
        in_specs=[
            pl.BlockSpec((EB, DIM), lambda i, j: (j, 0)),
            pl.BlockSpec((1, 2 * DIM, K), lambda i, j: (i, 0, 0)),
            pl.BlockSpec((1, 1, K), lambda i, j: (i, 0, 0)),
        ],
        out_specs=pl.BlockSpec((1, EB, 16), lambda i, j: (i, j, 0)),
        out_shape=jax.ShapeDtypeStruct((3, EP, 16), F32),
    )(edge_attr_p, m, c)


def _prep0_body(x_ref, w_ref, b_ref, ga_ref, gb_ref, rt_ref,
                hga_ref, hgb_ref, hrt_ref):
    h = jnp.dot(x_ref[...], w_ref[...], preferred_element_type=F32) + b_ref[...]
    hga_ref[...] = jnp.dot(h, ga_ref[...], preferred_element_type=F32)
    hgb_ref[...] = jnp.dot(h, gb_ref[...], preferred_element_type=F32)
    hrt_ref[...] = jnp.dot(h, rt_ref[...], preferred_element_type=F32)


def _prep0_call(x, w, b, ga, gb, rt):
    return pl.pallas_call(
        _prep0_body,
        grid=(N // NB,),
        in_specs=[
            pl.BlockSpec((NB, F_IN), lambda i: (i, 0)),
            pl.BlockSpec((F_IN, H), lambda i: (0, 0)),
            pl.BlockSpec((1, H), lambda i: (0, 0)),
            pl.BlockSpec((H, KH), lambda i: (0, 0)),
            pl.BlockSpec((H, KH), lambda i: (0, 0)),
            pl.BlockSpec((H, H), lambda i: (0, 0)),
        ],
        out_specs=[
            pl.BlockSpec((NB, KH), lambda i: (i, 0)),
            pl.BlockSpec((NB, KH), lambda i: (i, 0)),
            pl.BlockSpec((NB, H), lambda i: (i, 0)),
        ],
        out_shape=[
            jax.ShapeDtypeStruct((N, KH), F32),
            jax.ShapeDtypeStruct((N, KH), F32),
            jax.ShapeDtypeStruct((N, H), F32),
        ],
    )(x, w, b, ga, gb, rt)


def _combine(aggp_ref, degp_ref, hrt_ref, bias_ref):
    deg = degp_ref[0, :, :1] + degp_ref[1, :, :1]
    deg = jnp.maximum(deg, 1.0)
    a0 = aggp_ref[0, 0] + aggp_ref[1, 0]
    a1 = aggp_ref[0, 1] + aggp_ref[1, 1]
    agg = jnp.concatenate([a0, a1], axis=1)
    return agg / deg + hrt_ref[...] + bias_ref[...]


def _prep_mid_body(aggp_ref, degp_ref, hrt_ref, bias_ref,
                   ga_ref, gb_ref, rt_ref, hga_ref, hgb_ref, hrt_o_ref):
    ht = jnp.tanh(_combine(aggp_ref, degp_ref, hrt_ref, bias_ref))
    hga_ref[...] = jnp.dot(ht, ga_ref[...], preferred_element_type=F32)
    hgb_ref[...] = jnp.dot(ht, gb_ref[...], preferred_element_type=F32)
    hrt_o_ref[...] = jnp.dot(ht, rt_ref[...], preferred_element_type=F32)


def _prep_mid_call(aggp, degp, hrt, bias, ga, gb, rt):
    return pl.pallas_call(
        _prep_mid_body,
        grid=(N // NB,),
        in_specs=[
            pl.BlockSpec((NC, 2, NB, HH), lambda i: (0, 0, i, 0)),
            pl.BlockSpec((NC, NB, HH), lambda i: (0, i, 0)),
            pl.BlockSpec((NB, H), lambda i: (i, 0)),
            pl.BlockSpec((1, H), lambda i: (0, 0)),
            pl.BlockSpec((H, KH), lambda i: (0, 0)),
            pl.BlockSpec((H, KH), lambda i: (0, 0)),
            pl.BlockSpec((H, H), lambda i: (0, 0)),
        ],
        out_specs=[
            pl.BlockSpec((NB, KH), lambda i: (i, 0)),
            pl.BlockSpec((NB, KH), lambda i: (i, 0)),
            pl.BlockSpec((NB, H), lambda i: (i, 0)),
        ],
        out_shape=[
            jax.ShapeDtypeStruct((N, KH), F32),
            jax.ShapeDtypeStruct((N, KH), F32),
            jax.ShapeDtypeStruct((N, H), F32),
        ],
    )(aggp, degp, hrt, bias, ga, gb, rt)


def _pool_body(aggp_ref, degp_ref, hrt_ref, bias_ref, batch_ref, out_ref):
    i = pl.program_id(0)
    h3 = _combine(aggp_ref, degp_ref, hrt_ref, bias_ref)
    b = batch_ref[0, 0]
    onehot = (lax.broadcasted_iota(jnp.int32, (BG, NB), 0)
              == b[None, :]).astype(F32)
    part = jnp.dot(onehot, h3, preferred_element_type=F32)

    @pl.when(i == 0)
    def _():
        out_ref[...] = jnp.zeros_like(out_ref)
    out_ref[...] += part


def _pool_call(aggp, degp, hrt, bias, batch3):
    return pl.pallas_call(
        _pool_body,
        grid=(N // NB,),
        in_specs=[
            pl.BlockSpec((NC, 2, NB, HH), lambda i: (0, 0, i, 0)),
            pl.BlockSpec((NC, NB, HH), lambda i: (0, i, 0)),
            pl.BlockSpec((NB, H), lambda i: (i, 0)),
            pl.BlockSpec((1, H), lambda i: (0, 0)),
            pl.BlockSpec((1, 1, NB), lambda i: (i, 0, 0)),
        ],
        out_specs=pl.BlockSpec((BG, H), lambda i: (0, 0)),
        out_shape=jax.ShapeDtypeStruct((BG, H), F32),
    )(aggp, degp, hrt, bias, batch3)


def _bn(h, gamma, beta):
    m = jnp.mean(h, axis=0, keepdims=True)
    v = jnp.mean((h - m) ** 2, axis=0, keepdims=True)
    return gamma * (h - m) * lax.rsqrt(v + 1e-5) + beta


def _head_body(pooled_ref, lw_ref, lb_ref, gam_ref, bet_ref, wo_ref, bo_ref,
               out_ref):
    h = pooled_ref[...]
    for i in range(3):
        h = _bn(h, gam_ref[i], bet_ref[i])
        h = jnp.dot(h, lw_ref[i], preferred_element_type=F32) + lb_ref[i]
        h = jnp.tanh(h)
    h = _bn(h, gam_ref[3], bet_ref[3])
    out_ref[...] = jnp.dot(h, wo_ref[...], preferred_element_type=F32) + bo_ref[...]


def _head_call(pooled, lw, lb, gam, bet, wo, bo):
    full = lambda *s: pl.BlockSpec(s, lambda: tuple(0 for _ in s))
    return pl.pallas_call(
        _head_body,
        in_specs=[
            full(BG, H),
            full(3, H, H),
            full(3, 1, H),
            full(4, 1, H),
            full(4, 1, H),
            full(H, OUT),
            full(1, OUT),
        ],
        out_specs=full(BG, OUT),
        out_shape=jax.ShapeDtypeStruct((BG, OUT), F32),
    )(pooled, lw, lb, gam, bet, wo, bo)


def kernel(x, edge_attr, params, edge_index, batch):
    p = params
    src = edge_index[0]
    dst = edge_index[1]

    mu = p["conv_mu"]
    sg = p["conv_sigma"]
    a = 1.0 / (1e-15 + sg * sg)
    m = jnp.concatenate([(-0.5 * a).transpose(0, 2, 1),
                         (mu * a).transpose(0, 2, 1)], axis=1)
    cc = (-0.5 * jnp.sum(mu * mu * a, axis=-1))[:, None, :]
    eap = jnp.concatenate([edge_attr, jnp.zeros((EP - E, DIM), F32)])
    gauss = _gauss_call(eap, m, cc)

    g4 = p["conv_g"].reshape(3, H, K, H)
    ga = g4[:, :, :, :HH].reshape(3, H, KH)
    gb = g4[:, :, :, HH:].reshape(3, H, KH)

    zrow = jnp.zeros((N, HH), F32)
    pad = EP - E
    src3 = jnp.concatenate([src, jnp.zeros((pad,), jnp.int32)]
                           ).reshape(EP // CH, 1, CH)
    dst3 = jnp.concatenate([dst, jnp.full((pad,), N, jnp.int32)]
                           ).reshape(EP // CH, 1, CH)
    gauss3 = gauss.reshape(3, EP // CH, CH, 16)

    hga, hgb, hrt = _prep0_call(x, p["W_ne"], p["b_ne"].reshape(1, H),
                                ga[0], gb[0], p["conv_root"][0])
    aggp, degp = _sc_conv_deg(src3, dst3, gauss3[0], hga, hgb, zrow)

    for i in (1, 2):
        hga, hgb, hrt = _prep_mid_call(
            aggp, degp, hrt, p["conv_bias"][i - 1].reshape(1, H),
            ga[i], gb[i], p["conv_root"][i])
        aggp = _sc_conv(src3, dst3, gauss3[i], hga, hgb, zrow)

    pooled = _pool_call(aggp, degp, hrt, p["conv_bias"][2].reshape(1, H),
                        batch.reshape(N // NB, 1, NB))

    return _head_call(pooled, p["lin_w"], p["lin_b"].reshape(3, 1, H),
                      p["bn_gamma"].reshape(4, 1, H),
                      p["bn_beta"].reshape(4, 1, H),
                      p["W_out"], p["b_out"].reshape(1, OUT))
